# Initial kernel scaffold; baseline (speedup 1.0000x reference)
import jax
import jax.numpy as jnp
from jax.experimental import pallas as pl


def kernel(x, k):
    raise NotImplementedError("write your pallas kernel here")


# speedup vs baseline: 1.2825x; 1.2825x over previous
import jax
import jax.numpy as jnp
import numpy as np
from jax.experimental import pallas as pl

_K = 32
_SIGN = np.int32(-2147483648)


def _monotone_key(xb):
    bits = jax.lax.bitcast_convert_type(xb, jnp.int32)
    return jnp.where(bits >= 0, bits, -(bits ^ _SIGN))


def _body(x_ref, o_ref):
    xb = x_ref[...]
    skey = _monotone_key(xb)
    rows = xb.shape[0]

    def t_step(i, prefix_u):
        bit = (jnp.uint32(2147483648) >> i).astype(jnp.int32)
        cand_u = prefix_u | bit
        s_cand = cand_u ^ _SIGN
        cnt = jnp.sum((skey >= s_cand).astype(jnp.int32), axis=1, keepdims=True)
        return jnp.where(cnt >= _K, cand_u, prefix_u)

    t_u = jax.lax.fori_loop(0, 32, t_step, jnp.zeros((rows, 1), jnp.int32))
    t_s = t_u ^ _SIGN

    gt = skey > t_s
    eq = skey == t_s
    r = _K - jnp.sum(gt.astype(jnp.int32), axis=1, keepdims=True)

    col = jax.lax.broadcasted_iota(jnp.int32, xb.shape, 1)

    def i_step(b, ans):
        cand = ans | (jnp.int32(1) << (14 - b))
        cntl = jnp.sum((eq & (col < cand)).astype(jnp.int32), axis=1,
                       keepdims=True)
        return jnp.where(cntl < r, cand, ans)

    i_r = jax.lax.fori_loop(0, 15, i_step, jnp.zeros((rows, 1), jnp.int32))

    keep = gt | (eq & (col <= i_r))
    o_ref[...] = jnp.where(keep, jnp.maximum(xb, 0.0), 0.0)


def kernel(x, k):
    del k
    R, N = x.shape
    BR = 8
    return pl.pallas_call(
        _body,
        grid=(R // BR,),
        in_specs=[pl.BlockSpec((BR, N), lambda i: (i, 0))],
        out_specs=pl.BlockSpec((BR, N), lambda i: (i, 0)),
        out_shape=jax.ShapeDtypeStruct(x.shape, x.dtype),
    )(x)


# speedup vs baseline: 2.0600x; 1.6062x over previous
import functools

import jax
import jax.numpy as jnp
import numpy as np
from jax import lax
from jax.experimental import pallas as pl
from jax.experimental.pallas import tpu as pltpu
from jax.experimental.pallas import tpu_sc as plsc

_K = 32
_BIG_IDX = np.int32(2147483647)

_R, _N = 128, 32768
_NC, _NS = 2, 16
_NW = _NC * _NS
_RPW = _R // _NW
_NV = _N // 16


def _sc_body(x_hbm, out_hbm, row_v, valb, idxb, res_v):
    wid = lax.axis_index("s") * _NC + lax.axis_index("c")
    iota = lax.iota(jnp.int32, 16)

    def row_loop(j, res):
        r = wid * _RPW + j
        pltpu.sync_copy(x_hbm.at[r], row_v)

        def p1(i, accs):
            a0, a1 = accs
            v0 = row_v[pl.ds(i * 32, 16)]
            v1 = row_v[pl.ds(i * 32 + 16, 16)]
            return jnp.maximum(a0, v0), jnp.maximum(a1, v1)

        ninf = jnp.full((16,), -jnp.inf, jnp.float32)
        a0, a1 = lax.fori_loop(0, _N // 32, p1, (ninf, ninf))
        s = -jnp.max(-jnp.minimum(a0, a1))
        svec = jnp.full((16,), s, jnp.float32)

        def p2(i, cnt):
            v = row_v[pl.ds(i * 16, 16)]
            msk = v >= svec
            plsc.store_compressed(valb.at[pl.ds(cnt, 16)], v, mask=msk)
            plsc.store_compressed(idxb.at[pl.ds(cnt, 16)], iota + i * 16,
                                  mask=msk)
            return cnt + jnp.max(plsc.all_reduce_population_count(msk))

        cnt = lax.fori_loop(0, _NV, p2, jnp.int32(0))
        valb[pl.ds(cnt, 16)] = ninf
        idxb[pl.ds(cnt, 16)] = jnp.full((16,), _BIG_IDX, jnp.int32)
        nvq = (cnt + 15) // 16

        def ext(e, carry):
            del e, carry

            def fmax(q, acc):
                return jnp.maximum(acc, valb[pl.ds(q * 16, 16)])

            mk = lax.fori_loop(0, nvq, fmax, ninf)
            km = jnp.max(mk)
            kmv = jnp.full((16,), km, jnp.float32)

            def fmin(q, acc):
                vq = valb[pl.ds(q * 16, 16)]
                iq = idxb[pl.ds(q * 16, 16)]
                return jnp.minimum(acc, jnp.where(vq == kmv, iq, _BIG_IDX))

            mi = lax.fori_loop(0, nvq, fmin,
                               jnp.full((16,), _BIG_IDX, jnp.int32))
            im = -jnp.max(-mi)
            imv = jnp.full((16,), im, jnp.int32)

            def frem(q, _):
                vq = valb[pl.ds(q * 16, 16)]
                iq = idxb[pl.ds(q * 16, 16)]
                hit = (vq == kmv) & (iq == imv)
                valb[pl.ds(q * 16, 16)] = jnp.where(hit, ninf, vq)
                idxb[pl.ds(q * 16, 16)] = jnp.where(
                    hit, jnp.full((16,), _BIG_IDX, jnp.int32), iq)
                return 0

            lax.fori_loop(0, nvq, frem, 0)
            return km, im

        tval, tidx = lax.fori_loop(0, _K, ext,
                                   (jnp.float32(0), jnp.int32(0)))
        tbits = lax.bitcast_convert_type(tval, jnp.int32)
        res = jnp.where(iota == 2 * j, jnp.full((16,), tbits, jnp.int32), res)
        res = jnp.where(iota == 2 * j + 1, jnp.full((16,), tidx, jnp.int32),
                        res)
        return res

    res = lax.fori_loop(0, _RPW, row_loop, jnp.zeros((16,), jnp.int32))
    res_v[...] = res
    pltpu.sync_copy(res_v, out_hbm.at[wid])


_sc_thresholds = functools.partial(
    pl.kernel,
    out_type=jax.ShapeDtypeStruct((_NW, 16), jnp.int32),
    mesh=plsc.VectorSubcoreMesh(core_axis_name="c", subcore_axis_name="s"),
    compiler_params=pltpu.CompilerParams(needs_layout_passes=False),
    scratch_types=[
        pltpu.VMEM((_N,), jnp.float32),
        pltpu.VMEM((_N + 16,), jnp.float32),
        pltpu.VMEM((_N + 16,), jnp.int32),
        pltpu.VMEM((16,), jnp.int32),
    ],
)(_sc_body)


def _mask_body(x_ref, t_ref, i_ref, o_ref):
    xb = x_ref[...]
    t = t_ref[...]
    ir = i_ref[...]
    col = lax.broadcasted_iota(jnp.int32, xb.shape, 1)
    keep = (xb > t) | ((xb == t) & (col <= ir))
    o_ref[...] = jnp.where(keep, jnp.maximum(xb, 0.0), 0.0)


def kernel(x, k):
    del k
    packed = _sc_thresholds(x)
    q = packed[:, :8].reshape(_R, 2)
    tf = lax.bitcast_convert_type(q[:, 0], jnp.float32).reshape(_R, 1)
    ir = q[:, 1].reshape(_R, 1)

    BR = 8
    return pl.pallas_call(
        _mask_body,
        grid=(_R // BR,),
        in_specs=[
            pl.BlockSpec((BR, _N), lambda i: (i, 0)),
            pl.BlockSpec((BR, 1), lambda i: (i, 0)),
            pl.BlockSpec((BR, 1), lambda i: (i, 0)),
        ],
        out_specs=pl.BlockSpec((BR, _N), lambda i: (i, 0)),
        out_shape=jax.ShapeDtypeStruct(x.shape, x.dtype),
    )(x, tf, ir)


# speedup vs baseline: 2.1398x; 1.0387x over previous
import functools

import jax
import jax.numpy as jnp
import numpy as np
from jax import lax
from jax.experimental import pallas as pl
from jax.experimental.pallas import tpu as pltpu
from jax.experimental.pallas import tpu_sc as plsc

_K = 32
_BIG_IDX = np.int32(2147483647)

_R, _N = 128, 32768
_NC, _NS = 2, 16
_NW = _NC * _NS
_RPW = _R // _NW
_NV = _N // 16


def _sc_body(x_hbm, out_hbm, row_v, idxb, res_v):
    wid = lax.axis_index("s") * _NC + lax.axis_index("c")
    iota = lax.iota(jnp.int32, 16)
    ninf = jnp.full((16,), -jnp.inf, jnp.float32)

    def row_loop(j, res):
        r = wid * _RPW + j
        pltpu.sync_copy(x_hbm.at[r], row_v.at[pl.ds(0, _N)])
        row_v[pl.ds(_N, 16)] = ninf

        def p1(i, accs):
            return tuple(
                jnp.maximum(accs[u], row_v[pl.ds(i * 128 + u * 16, 16)])
                for u in range(8))

        accs = lax.fori_loop(0, _N // 128, p1, (ninf,) * 8)
        m0 = jnp.maximum(jnp.maximum(accs[0], accs[1]),
                         jnp.maximum(accs[2], accs[3]))
        m1 = jnp.maximum(jnp.maximum(accs[4], accs[5]),
                         jnp.maximum(accs[6], accs[7]))
        s = -jnp.max(jnp.maximum(-m0, -m1))
        svec = jnp.full((16,), s, jnp.float32)

        def p2(i, cnt):
            v = row_v[pl.ds(i * 16, 16)]
            msk = v >= svec
            plsc.store_compressed(idxb.at[pl.ds(cnt, 16)], iota + i * 16,
                                  mask=msk)
            return cnt + jnp.max(plsc.all_reduce_population_count(msk))

        cnt = lax.fori_loop(0, _NV, p2, jnp.int32(0))
        idxb[pl.ds(cnt, 16)] = jnp.full((16,), _N, jnp.int32)
        nvq = (cnt + 15) // 16

        def ext(e, carry):
            del e, carry

            def fmax(q, acc):
                iq = idxb[pl.ds(q * 16, 16)]
                vq = plsc.load_gather(row_v, [iq])
                return jnp.maximum(acc, vq)

            mk = lax.fori_loop(0, nvq, fmax, ninf)
            km = jnp.max(mk)
            kmv = jnp.full((16,), km, jnp.float32)

            def fmin(q, acc):
                iq = idxb[pl.ds(q * 16, 16)]
                vq = plsc.load_gather(row_v, [iq])
                return jnp.minimum(acc, jnp.where(vq == kmv, iq, _BIG_IDX))

            mi = lax.fori_loop(0, nvq, fmin,
                               jnp.full((16,), _BIG_IDX, jnp.int32))
            im = -jnp.max(-mi)
            imv = jnp.full((16,), im, jnp.int32)

            def frem(q, _):
                iq = idxb[pl.ds(q * 16, 16)]
                idxb[pl.ds(q * 16, 16)] = jnp.where(
                    iq == imv, jnp.full((16,), _N, jnp.int32), iq)
                return 0

            lax.fori_loop(0, nvq, frem, 0)
            return km, im

        tval, tidx = lax.fori_loop(0, _K, ext,
                                   (jnp.float32(0), jnp.int32(0)))
        tbits = lax.bitcast_convert_type(tval, jnp.int32)
        res = jnp.where(iota == 2 * j, jnp.full((16,), tbits, jnp.int32), res)
        res = jnp.where(iota == 2 * j + 1, jnp.full((16,), tidx, jnp.int32),
                        res)
        return res

    res = lax.fori_loop(0, _RPW, row_loop, jnp.zeros((16,), jnp.int32))
    res_v[...] = res
    pltpu.sync_copy(res_v, out_hbm.at[wid])


_sc_thresholds = functools.partial(
    pl.kernel,
    out_type=jax.ShapeDtypeStruct((_NW, 16), jnp.int32),
    mesh=plsc.VectorSubcoreMesh(core_axis_name="c", subcore_axis_name="s"),
    compiler_params=pltpu.CompilerParams(needs_layout_passes=False),
    scratch_types=[
        pltpu.VMEM((_N + 16,), jnp.float32),
        pltpu.VMEM((_N + 16,), jnp.int32),
        pltpu.VMEM((16,), jnp.int32),
    ],
)(_sc_body)


def _mask_body(x_ref, t_ref, i_ref, o_ref):
    xb = x_ref[...]
    t = t_ref[...]
    ir = i_ref[...]
    col = lax.broadcasted_iota(jnp.int32, xb.shape, 1)
    keep = (xb > t) | ((xb == t) & (col <= ir))
    o_ref[...] = jnp.where(keep, jnp.maximum(xb, 0.0), 0.0)


def kernel(x, k):
    del k
    packed = _sc_thresholds(x)
    q = packed[:, :8].reshape(_R, 2)
    tf = lax.bitcast_convert_type(q[:, 0], jnp.float32).reshape(_R, 1)
    ir = q[:, 1].reshape(_R, 1)

    BR = 8
    return pl.pallas_call(
        _mask_body,
        grid=(_R // BR,),
        in_specs=[
            pl.BlockSpec((BR, _N), lambda i: (i, 0)),
            pl.BlockSpec((BR, 1), lambda i: (i, 0)),
            pl.BlockSpec((BR, 1), lambda i: (i, 0)),
        ],
        out_specs=pl.BlockSpec((BR, _N), lambda i: (i, 0)),
        out_shape=jax.ShapeDtypeStruct(x.shape, x.dtype),
    )(x, tf, ir)


# speedup vs baseline: 2.7755x; 1.2971x over previous
import functools

import jax
import jax.numpy as jnp
import numpy as np
from jax import lax
from jax.experimental import pallas as pl
from jax.experimental.pallas import tpu as pltpu
from jax.experimental.pallas import tpu_sc as plsc

_K = 32
_BIG_IDX = np.int32(2147483647)

_R, _N = 128, 32768
_NC, _NS = 2, 16
_NW = _NC * _NS
_RPW = _R // _NW
_NV = _N // 16


def _sc_body(x_hbm, out_hbm, row_v, idxb, gmax_s, res_v):
    wid = lax.axis_index("s") * _NC + lax.axis_index("c")
    iota = lax.iota(jnp.int32, 16)
    ninf = jnp.full((16,), -jnp.inf, jnp.float32)

    def row_loop(j, res):
        r = wid * _RPW + j
        pltpu.sync_copy(x_hbm.at[r], row_v.at[pl.ds(0, _N)])
        row_v[pl.ds(_N, 16)] = ninf

        def p1(g, accs):
            vs = [row_v[pl.ds(g * 128 + u * 16, 16)] for u in range(8)]
            f01 = jnp.maximum(vs[0], vs[1])
            f23 = jnp.maximum(vs[2], vs[3])
            f45 = jnp.maximum(vs[4], vs[5])
            f67 = jnp.maximum(vs[6], vs[7])
            gmax_s[g] = jnp.max(jnp.maximum(jnp.maximum(f01, f23),
                                            jnp.maximum(f45, f67)))
            return tuple(jnp.maximum(accs[u], vs[u]) for u in range(8))

        accs = lax.fori_loop(0, _N // 128, p1, (ninf,) * 8)
        m0 = jnp.maximum(jnp.maximum(accs[0], accs[1]),
                         jnp.maximum(accs[2], accs[3]))
        m1 = jnp.maximum(jnp.maximum(accs[4], accs[5]),
                         jnp.maximum(accs[6], accs[7]))
        s = -jnp.max(jnp.maximum(-m0, -m1))
        svec = jnp.full((16,), s, jnp.float32)

        def collect8(g, cntv):
            def c1(u, cv):
                v = row_v[pl.ds(g * 128 + u * 16, 16)]
                msk = v >= svec
                plsc.store_compressed(
                    idxb.at[pl.ds(cv[0], 16)],
                    iota + (g * 128 + u * 16), mask=msk)
                return cv + plsc.all_reduce_population_count(msk)

            return lax.fori_loop(0, 8, c1, cntv)

        def p2(g, cntv):
            return lax.cond(gmax_s[g] >= s, lambda cv: collect8(g, cv),
                            lambda cv: cv, cntv)

        cntv = lax.fori_loop(0, _N // 128, p2,
                             jnp.zeros((16,), jnp.int32))
        cnt = cntv[0]
        idxb[pl.ds(cnt, 16)] = jnp.full((16,), _N, jnp.int32)
        nvq = (cnt + 15) // 16

        def ext(e, carry):
            del e, carry

            def fmax(q, acc):
                iq = idxb[pl.ds(q * 16, 16)]
                vq = plsc.load_gather(row_v, [iq])
                return jnp.maximum(acc, vq)

            mk = lax.fori_loop(0, nvq, fmax, ninf)
            km = jnp.max(mk)
            kmv = jnp.full((16,), km, jnp.float32)

            def fmin(q, acc):
                iq = idxb[pl.ds(q * 16, 16)]
                vq = plsc.load_gather(row_v, [iq])
                return jnp.minimum(acc, jnp.where(vq == kmv, iq, _BIG_IDX))

            mi = lax.fori_loop(0, nvq, fmin,
                               jnp.full((16,), _BIG_IDX, jnp.int32))
            im = -jnp.max(-mi)
            imv = jnp.full((16,), im, jnp.int32)

            def frem(q, _):
                iq = idxb[pl.ds(q * 16, 16)]
                idxb[pl.ds(q * 16, 16)] = jnp.where(
                    iq == imv, jnp.full((16,), _N, jnp.int32), iq)
                return 0

            lax.fori_loop(0, nvq, frem, 0)
            return km, im

        tval, tidx = lax.fori_loop(0, _K, ext,
                                   (jnp.float32(0), jnp.int32(0)))
        tbits = lax.bitcast_convert_type(tval, jnp.int32)
        res = jnp.where(iota == 2 * j, jnp.full((16,), tbits, jnp.int32), res)
        res = jnp.where(iota == 2 * j + 1, jnp.full((16,), tidx, jnp.int32),
                        res)
        return res

    res = lax.fori_loop(0, _RPW, row_loop, jnp.zeros((16,), jnp.int32))
    res_v[...] = res
    pltpu.sync_copy(res_v, out_hbm.at[wid])


_sc_thresholds = functools.partial(
    pl.kernel,
    out_type=jax.ShapeDtypeStruct((_NW, 16), jnp.int32),
    mesh=plsc.VectorSubcoreMesh(core_axis_name="c", subcore_axis_name="s"),
    compiler_params=pltpu.CompilerParams(needs_layout_passes=False),
    scratch_types=[
        pltpu.VMEM((_N + 16,), jnp.float32),
        pltpu.VMEM((_N + 16,), jnp.int32),
        pltpu.SMEM((_N // 128,), jnp.float32),
        pltpu.VMEM((16,), jnp.int32),
    ],
)(_sc_body)


def _mask_body(x_ref, t_ref, i_ref, o_ref):
    xb = x_ref[...]
    t = t_ref[...]
    ir = i_ref[...]
    col = lax.broadcasted_iota(jnp.int32, xb.shape, 1)
    keep = (xb > t) | ((xb == t) & (col <= ir))
    o_ref[...] = jnp.where(keep, jnp.maximum(xb, 0.0), 0.0)


def kernel(x, k):
    del k
    packed = _sc_thresholds(x)
    q = packed[:, :8].reshape(_R, 2)
    tf = lax.bitcast_convert_type(q[:, 0], jnp.float32).reshape(_R, 1)
    ir = q[:, 1].reshape(_R, 1)

    BR = 8
    return pl.pallas_call(
        _mask_body,
        grid=(_R // BR,),
        in_specs=[
            pl.BlockSpec((BR, _N), lambda i: (i, 0)),
            pl.BlockSpec((BR, 1), lambda i: (i, 0)),
            pl.BlockSpec((BR, 1), lambda i: (i, 0)),
        ],
        out_specs=pl.BlockSpec((BR, _N), lambda i: (i, 0)),
        out_shape=jax.ShapeDtypeStruct(x.shape, x.dtype),
    )(x, tf, ir)


# speedup vs baseline: 2.8600x; 1.0304x over previous
import functools

import jax
import jax.numpy as jnp
import numpy as np
from jax import lax
from jax.experimental import pallas as pl
from jax.experimental.pallas import tpu as pltpu
from jax.experimental.pallas import tpu_sc as plsc

_K = 32
_BIG_IDX = np.int32(2147483647)

_R, _N = 128, 32768
_NC, _NS = 2, 16
_NW = _NC * _NS
_RPW = _R // _NW
_NV = _N // 16


def _sc_body(x_hbm, out_hbm, buf0, buf1, idxb, gmax_s, res_v, sem0, sem1,
             rsem):
    wid = lax.axis_index("s") * _NC + lax.axis_index("c")
    base = wid * _RPW
    iota = lax.iota(jnp.int32, 16)
    ninf = jnp.full((16,), -jnp.inf, jnp.float32)

    bufs = (buf0, buf1)
    sems = (sem0, sem1)
    buf0[pl.ds(_N, 16)] = ninf
    buf1[pl.ds(_N, 16)] = ninf

    pending = pltpu.async_copy(x_hbm.at[base], buf0.at[pl.ds(0, _N)], sem0)
    res = jnp.zeros((16,), jnp.int32)

    for j in range(_RPW):
        row_v = bufs[j % 2]
        pending.wait()
        if j + 1 < _RPW:
            pending = pltpu.async_copy(
                x_hbm.at[base + j + 1],
                bufs[(j + 1) % 2].at[pl.ds(0, _N)], sems[(j + 1) % 2])

        def p1(g, accs, row_v=row_v):
            vs = [row_v[pl.ds(g * 128 + u * 16, 16)] for u in range(8)]
            f01 = jnp.maximum(vs[0], vs[1])
            f23 = jnp.maximum(vs[2], vs[3])
            f45 = jnp.maximum(vs[4], vs[5])
            f67 = jnp.maximum(vs[6], vs[7])
            gmax_s[g] = jnp.max(jnp.maximum(jnp.maximum(f01, f23),
                                            jnp.maximum(f45, f67)))
            return tuple(jnp.maximum(accs[u], vs[u]) for u in range(8))

        accs = lax.fori_loop(0, _N // 128, p1, (ninf,) * 8)
        m0 = jnp.maximum(jnp.maximum(accs[0], accs[1]),
                         jnp.maximum(accs[2], accs[3]))
        m1 = jnp.maximum(jnp.maximum(accs[4], accs[5]),
                         jnp.maximum(accs[6], accs[7]))
        s = -jnp.max(jnp.maximum(-m0, -m1))
        svec = jnp.full((16,), s, jnp.float32)

        def collect8(g, cntv, row_v=row_v, svec=svec):
            def c1(u, cv):
                v = row_v[pl.ds(g * 128 + u * 16, 16)]
                msk = v >= svec
                plsc.store_compressed(
                    idxb.at[pl.ds(cv[0], 16)],
                    iota + (g * 128 + u * 16), mask=msk)
                return cv + plsc.all_reduce_population_count(msk)

            return lax.fori_loop(0, 8, c1, cntv)

        def p2(g, cntv, s=s, collect8=collect8):
            return lax.cond(gmax_s[g] >= s, lambda cv: collect8(g, cv),
                            lambda cv: cv, cntv)

        cntv = lax.fori_loop(0, _N // 128, p2, jnp.zeros((16,), jnp.int32))
        cnt = cntv[0]
        idxb[pl.ds(cnt, 16)] = jnp.full((16,), _N, jnp.int32)
        nvq = (cnt + 15) // 16

        def ext(e, carry, row_v=row_v, nvq=nvq):
            del e, carry

            def fmax(q, acc):
                iq = idxb[pl.ds(q * 16, 16)]
                vq = plsc.load_gather(row_v, [iq])
                return jnp.maximum(acc, vq)

            mk = lax.fori_loop(0, nvq, fmax, ninf)
            km = jnp.max(mk)
            kmv = jnp.full((16,), km, jnp.float32)

            def fmin(q, acc):
                iq = idxb[pl.ds(q * 16, 16)]
                vq = plsc.load_gather(row_v, [iq])
                return jnp.minimum(acc, jnp.where(vq == kmv, iq, _BIG_IDX))

            mi = lax.fori_loop(0, nvq, fmin,
                               jnp.full((16,), _BIG_IDX, jnp.int32))
            im = -jnp.max(-mi)
            imv = jnp.full((16,), im, jnp.int32)

            def frem(q, _):
                iq = idxb[pl.ds(q * 16, 16)]
                idxb[pl.ds(q * 16, 16)] = jnp.where(
                    iq == imv, jnp.full((16,), _N, jnp.int32), iq)
                return 0

            lax.fori_loop(0, nvq, frem, 0)
            return km, im

        tval, tidx = lax.fori_loop(0, _K, ext,
                                   (jnp.float32(0), jnp.int32(0)))
        tbits = lax.bitcast_convert_type(tval, jnp.int32)
        res = jnp.where(iota == 2 * j, jnp.full((16,), tbits, jnp.int32),
                        res)
        res = jnp.where(iota == 2 * j + 1, jnp.full((16,), tidx, jnp.int32),
                        res)

    res_v[...] = res
    pltpu.async_copy(res_v, out_hbm.at[wid], rsem).wait()


_sc_thresholds = functools.partial(
    pl.kernel,
    out_type=jax.ShapeDtypeStruct((_NW, 16), jnp.int32),
    mesh=plsc.VectorSubcoreMesh(core_axis_name="c", subcore_axis_name="s"),
    compiler_params=pltpu.CompilerParams(needs_layout_passes=False),
    scratch_types=[
        pltpu.VMEM((_N + 16,), jnp.float32),
        pltpu.VMEM((_N + 16,), jnp.float32),
        pltpu.VMEM((_N + 16,), jnp.int32),
        pltpu.SMEM((_N // 128,), jnp.float32),
        pltpu.VMEM((16,), jnp.int32),
        pltpu.SemaphoreType.DMA,
        pltpu.SemaphoreType.DMA,
        pltpu.SemaphoreType.DMA,
    ],
)(_sc_body)


def _mask_body(x_ref, t_ref, i_ref, o_ref):
    xb = x_ref[...]
    t = t_ref[...]
    ir = i_ref[...]
    col = lax.broadcasted_iota(jnp.int32, xb.shape, 1)
    keep = (xb > t) | ((xb == t) & (col <= ir))
    o_ref[...] = jnp.where(keep, jnp.maximum(xb, 0.0), 0.0)


def kernel(x, k):
    del k
    packed = _sc_thresholds(x)
    q = packed[:, :8].reshape(_R, 2)
    tf = lax.bitcast_convert_type(q[:, 0], jnp.float32).reshape(_R, 1)
    ir = q[:, 1].reshape(_R, 1)

    BR = 8
    return pl.pallas_call(
        _mask_body,
        grid=(_R // BR,),
        in_specs=[
            pl.BlockSpec((BR, _N), lambda i: (i, 0)),
            pl.BlockSpec((BR, 1), lambda i: (i, 0)),
            pl.BlockSpec((BR, 1), lambda i: (i, 0)),
        ],
        out_specs=pl.BlockSpec((BR, _N), lambda i: (i, 0)),
        out_shape=jax.ShapeDtypeStruct(x.shape, x.dtype),
    )(x, tf, ir)


# speedup vs baseline: 4.8570x; 1.6982x over previous
import functools

import jax
import jax.numpy as jnp
import numpy as np
from jax import lax
from jax.experimental import pallas as pl
from jax.experimental.pallas import tpu as pltpu
from jax.experimental.pallas import tpu_sc as plsc

_K = 32
_BIG_IDX = np.int32(2147483647)

_R, _N = 128, 32768
_NC, _NS = 2, 16
_NW = _NC * _NS
_RPW = _R // _NW
_NV = _N // 16


def _sort16d(v):
    return -jnp.sort(-v)


def _merge32(a0, a1):
    rb = lax.rev(a1, (0,))
    hi = jnp.maximum(a0, rb)
    lo = jnp.minimum(a0, rb)
    return _sort16d(hi), _sort16d(lo)


def _top32_fold(acc, sv):
    a0, a1 = acc
    m0, _ = _merge32(a1, sv)
    return _merge32(a0, m0)


def _sc_body(x_hbm, out_hbm, buf0, buf1, idxb, gmax_s, res_v, sem0, sem1,
             rsem):
    wid = lax.axis_index("s") * _NC + lax.axis_index("c")
    base = wid * _RPW
    iota = lax.iota(jnp.int32, 16)
    ninf = jnp.full((16,), -jnp.inf, jnp.float32)

    bufs = (buf0, buf1)
    sems = (sem0, sem1)
    buf0[pl.ds(_N, 16)] = ninf
    buf1[pl.ds(_N, 16)] = ninf

    pending = pltpu.async_copy(x_hbm.at[base], buf0.at[pl.ds(0, _N)], sem0)
    res = jnp.zeros((16,), jnp.int32)

    for j in range(_RPW):
        row_v = bufs[j % 2]
        pending.wait()
        if j + 1 < _RPW:
            pending = pltpu.async_copy(
                x_hbm.at[base + j + 1],
                bufs[(j + 1) % 2].at[pl.ds(0, _N)], sems[(j + 1) % 2])

        def p1(g, accs, row_v=row_v):
            vs = [row_v[pl.ds(g * 128 + u * 16, 16)] for u in range(8)]
            f01 = jnp.maximum(vs[0], vs[1])
            f23 = jnp.maximum(vs[2], vs[3])
            f45 = jnp.maximum(vs[4], vs[5])
            f67 = jnp.maximum(vs[6], vs[7])
            gmax_s[g] = jnp.max(jnp.maximum(jnp.maximum(f01, f23),
                                            jnp.maximum(f45, f67)))
            return tuple(jnp.maximum(accs[u], vs[u]) for u in range(8))

        accs = lax.fori_loop(0, _N // 128, p1, (ninf,) * 8)
        sacc = _merge32(_sort16d(accs[0]), _sort16d(accs[1]))
        for u in range(2, 8):
            sacc = _top32_fold(sacc, _sort16d(accs[u]))
        s = sacc[1][15]
        svec = jnp.full((16,), s, jnp.float32)

        def collect8(g, cntv, row_v=row_v, svec=svec):
            def c1(u, cv):
                v = row_v[pl.ds(g * 128 + u * 16, 16)]
                msk = v >= svec
                plsc.store_compressed(
                    idxb.at[pl.ds(cv[0], 16)],
                    iota + (g * 128 + u * 16), mask=msk)
                return cv + plsc.all_reduce_population_count(msk)

            return lax.fori_loop(0, 8, c1, cntv)

        def p2(g, cntv, s=s, collect8=collect8):
            return lax.cond(gmax_s[g] >= s, lambda cv: collect8(g, cv),
                            lambda cv: cv, cntv)

        cntv = lax.fori_loop(0, _N // 128, p2, jnp.zeros((16,), jnp.int32))
        cnt = cntv[0]
        idxb[pl.ds(cnt, 16)] = jnp.full((16,), _N, jnp.int32)
        nvq = (cnt + 15) // 16

        def mstep(q, acc, row_v=row_v):
            iq = idxb[pl.ds(q * 16, 16)]
            vq = plsc.load_gather(row_v, [iq])
            return _top32_fold(acc, _sort16d(vq))

        acc0, acc1 = lax.fori_loop(0, nvq, mstep, (ninf, ninf))
        tval = acc1[15]
        tv = jnp.full((16,), tval, jnp.float32)

        def cgt(q, c, row_v=row_v):
            iq = idxb[pl.ds(q * 16, 16)]
            vq = plsc.load_gather(row_v, [iq])
            return c + plsc.all_reduce_population_count(vq > tv)

        r = _K - lax.fori_loop(0, nvq, cgt,
                               jnp.zeros((16,), jnp.int32))[0]

        def min_eq_idx(_, row_v=row_v, nvq=nvq):
            def fmin(q, acc):
                iq = idxb[pl.ds(q * 16, 16)]
                vq = plsc.load_gather(row_v, [iq])
                return jnp.minimum(acc, jnp.where(vq == tv, iq, _BIG_IDX))

            mi = lax.fori_loop(0, nvq, fmin,
                               jnp.full((16,), _BIG_IDX, jnp.int32))
            return -jnp.max(-mi)

        def rth_eq_idx(_, row_v=row_v, nvq=nvq, r=r):
            def bstep(b, ans):
                cand = ans | (jnp.int32(1) << (14 - b))
                cv = jnp.full((16,), cand, jnp.int32)

                def fcnt(q, c):
                    iq = idxb[pl.ds(q * 16, 16)]
                    vq = plsc.load_gather(row_v, [iq])
                    return c + plsc.all_reduce_population_count(
                        (vq == tv) & (iq < cv))

                cnt = lax.fori_loop(0, nvq, fcnt,
                                    jnp.zeros((16,), jnp.int32))[0]
                return jnp.where(cnt < r, cand, ans)

            return lax.fori_loop(0, 15, bstep, jnp.int32(0))

        tidx = lax.cond(r == 1, min_eq_idx, rth_eq_idx, 0)
        tbits = lax.bitcast_convert_type(tval, jnp.int32)
        res = jnp.where(iota == 2 * j, jnp.full((16,), tbits, jnp.int32),
                        res)
        res = jnp.where(iota == 2 * j + 1, jnp.full((16,), tidx, jnp.int32),
                        res)

    res_v[...] = res
    pltpu.async_copy(res_v, out_hbm.at[wid], rsem).wait()


_sc_thresholds = functools.partial(
    pl.kernel,
    out_type=jax.ShapeDtypeStruct((_NW, 16), jnp.int32),
    mesh=plsc.VectorSubcoreMesh(core_axis_name="c", subcore_axis_name="s"),
    compiler_params=pltpu.CompilerParams(needs_layout_passes=False),
    scratch_types=[
        pltpu.VMEM((_N + 16,), jnp.float32),
        pltpu.VMEM((_N + 16,), jnp.float32),
        pltpu.VMEM((_N + 16,), jnp.int32),
        pltpu.SMEM((_N // 128,), jnp.float32),
        pltpu.VMEM((16,), jnp.int32),
        pltpu.SemaphoreType.DMA,
        pltpu.SemaphoreType.DMA,
        pltpu.SemaphoreType.DMA,
    ],
)(_sc_body)


def _mask_body(x_ref, t_ref, i_ref, o_ref):
    xb = x_ref[...]
    t = t_ref[...]
    ir = i_ref[...]
    col = lax.broadcasted_iota(jnp.int32, xb.shape, 1)
    keep = (xb > t) | ((xb == t) & (col <= ir))
    o_ref[...] = jnp.where(keep, jnp.maximum(xb, 0.0), 0.0)


def kernel(x, k):
    del k
    packed = _sc_thresholds(x)
    q = packed[:, :8].reshape(_R, 2)
    tf = lax.bitcast_convert_type(q[:, 0], jnp.float32).reshape(_R, 1)
    ir = q[:, 1].reshape(_R, 1)

    BR = 8
    return pl.pallas_call(
        _mask_body,
        grid=(_R // BR,),
        in_specs=[
            pl.BlockSpec((BR, _N), lambda i: (i, 0)),
            pl.BlockSpec((BR, 1), lambda i: (i, 0)),
            pl.BlockSpec((BR, 1), lambda i: (i, 0)),
        ],
        out_specs=pl.BlockSpec((BR, _N), lambda i: (i, 0)),
        out_shape=jax.ShapeDtypeStruct(x.shape, x.dtype),
    )(x, tf, ir)


# speedup vs baseline: 5.2769x; 1.0865x over previous
import functools

import jax
import jax.numpy as jnp
import numpy as np
from jax import lax
from jax.experimental import pallas as pl
from jax.experimental.pallas import tpu as pltpu
from jax.experimental.pallas import tpu_sc as plsc

_K = 32
_BIG_IDX = np.int32(2147483647)

_R, _N = 128, 32768
_NC, _NS = 2, 16
_NW = _NC * _NS
_RPW = _R // _NW
_NV = _N // 16


def _sort16d(v):
    return -jnp.sort(-v)


def _merge32(a0, a1):
    rb = lax.rev(a1, (0,))
    hi = jnp.maximum(a0, rb)
    lo = jnp.minimum(a0, rb)
    return _sort16d(hi), _sort16d(lo)


def _top32_fold(acc, sv):
    a0, a1 = acc
    m0, _ = _merge32(a1, sv)
    return _merge32(a0, m0)


def _sc_body(x_hbm, out_hbm, buf0, buf1, idxb, gmax_s, res_v, sem0, sem1,
             rsem):
    wid = lax.axis_index("s") * _NC + lax.axis_index("c")
    base = wid * _RPW
    iota = lax.iota(jnp.int32, 16)
    ninf = jnp.full((16,), -jnp.inf, jnp.float32)

    bufs = (buf0, buf1)
    sems = (sem0, sem1)
    buf0[pl.ds(_N, 16)] = ninf
    buf1[pl.ds(_N, 16)] = ninf

    pending = pltpu.async_copy(x_hbm.at[base], buf0.at[pl.ds(0, _N)], sem0)
    res = jnp.zeros((16,), jnp.int32)

    for j in range(_RPW):
        row_v = bufs[j % 2]
        pending.wait()
        if j + 1 < _RPW:
            pending = pltpu.async_copy(
                x_hbm.at[base + j + 1],
                bufs[(j + 1) % 2].at[pl.ds(0, _N)], sems[(j + 1) % 2])

        def p1(g, accs, row_v=row_v):
            vs = [row_v[pl.ds(g * 128 + u * 16, 16)] for u in range(8)]
            f01 = jnp.maximum(vs[0], vs[1])
            f23 = jnp.maximum(vs[2], vs[3])
            f45 = jnp.maximum(vs[4], vs[5])
            f67 = jnp.maximum(vs[6], vs[7])
            gmax_s[g] = jnp.max(jnp.maximum(jnp.maximum(f01, f23),
                                            jnp.maximum(f45, f67)))
            return tuple(jnp.maximum(accs[u], vs[u]) for u in range(8))

        accs = lax.fori_loop(0, _N // 128, p1, (ninf,) * 8)
        sacc = _merge32(_sort16d(accs[0]), _sort16d(accs[1]))
        for u in range(2, 8):
            sacc = _top32_fold(sacc, _sort16d(accs[u]))
        s = sacc[1][15]
        svec = jnp.full((16,), s, jnp.float32)

        def collect8(g, cntv, row_v=row_v, svec=svec):
            def c1(u, cv):
                v = row_v[pl.ds(g * 128 + u * 16, 16)]
                msk = v >= svec
                plsc.store_compressed(
                    idxb.at[pl.ds(cv[0], 16)],
                    iota + (g * 128 + u * 16), mask=msk)
                return cv + plsc.all_reduce_population_count(msk)

            return lax.fori_loop(0, 8, c1, cntv)

        def p2(g, cntv, s=s, collect8=collect8):
            return lax.cond(gmax_s[g] >= s, lambda cv: collect8(g, cv),
                            lambda cv: cv, cntv)

        cntv = lax.fori_loop(0, _N // 128, p2, jnp.zeros((16,), jnp.int32))
        cnt = cntv[0]
        idxb[pl.ds(cnt, 16)] = jnp.full((16,), _N, jnp.int32)
        nvq = (cnt + 15) // 16

        def mstep(q, acc, row_v=row_v):
            iq = idxb[pl.ds(q * 16, 16)]
            vq = plsc.load_gather(row_v, [iq])
            return _top32_fold(acc, _sort16d(vq))

        acc0, acc1 = lax.fori_loop(0, nvq, mstep, (ninf, ninf))
        tval = acc1[15]
        tv = jnp.full((16,), tval, jnp.float32)

        def cgt(q, c, row_v=row_v):
            iq = idxb[pl.ds(q * 16, 16)]
            vq = plsc.load_gather(row_v, [iq])
            return c + plsc.all_reduce_population_count(vq > tv)

        r = _K - lax.fori_loop(0, nvq, cgt,
                               jnp.zeros((16,), jnp.int32))[0]

        def min_eq_idx(_, row_v=row_v, nvq=nvq):
            def fmin(q, acc):
                iq = idxb[pl.ds(q * 16, 16)]
                vq = plsc.load_gather(row_v, [iq])
                return jnp.minimum(acc, jnp.where(vq == tv, iq, _BIG_IDX))

            mi = lax.fori_loop(0, nvq, fmin,
                               jnp.full((16,), _BIG_IDX, jnp.int32))
            return -jnp.max(-mi)

        def rth_eq_idx(_, row_v=row_v, nvq=nvq, r=r):
            def bstep(b, ans):
                cand = ans | (jnp.int32(1) << (14 - b))
                cv = jnp.full((16,), cand, jnp.int32)

                def fcnt(q, c):
                    iq = idxb[pl.ds(q * 16, 16)]
                    vq = plsc.load_gather(row_v, [iq])
                    return c + plsc.all_reduce_population_count(
                        (vq == tv) & (iq < cv))

                cnt = lax.fori_loop(0, nvq, fcnt,
                                    jnp.zeros((16,), jnp.int32))[0]
                return jnp.where(cnt < r, cand, ans)

            return lax.fori_loop(0, 15, bstep, jnp.int32(0))

        tidx = lax.cond(r == 1, min_eq_idx, rth_eq_idx, 0)
        tbits = lax.bitcast_convert_type(tval, jnp.int32)
        res = jnp.where(iota == 2 * j, jnp.full((16,), tbits, jnp.int32),
                        res)
        res = jnp.where(iota == 2 * j + 1, jnp.full((16,), tidx, jnp.int32),
                        res)

    res_v[...] = res
    pltpu.async_copy(res_v, out_hbm.at[wid], rsem).wait()


_sc_thresholds = functools.partial(
    pl.kernel,
    out_type=jax.ShapeDtypeStruct((_NW, 16), jnp.int32),
    mesh=plsc.VectorSubcoreMesh(core_axis_name="c", subcore_axis_name="s"),
    compiler_params=pltpu.CompilerParams(needs_layout_passes=False),
    scratch_types=[
        pltpu.VMEM((_N + 16,), jnp.float32),
        pltpu.VMEM((_N + 16,), jnp.float32),
        pltpu.VMEM((_N + 16,), jnp.int32),
        pltpu.SMEM((_N // 128,), jnp.float32),
        pltpu.VMEM((16,), jnp.int32),
        pltpu.SemaphoreType.DMA,
        pltpu.SemaphoreType.DMA,
        pltpu.SemaphoreType.DMA,
    ],
)(_sc_body)


def _mask_body(x_ref, t_ref, i_ref, o_ref):
    xb = x_ref[...]
    t = t_ref[...]
    ir = i_ref[...]
    col = lax.broadcasted_iota(jnp.int32, xb.shape, 1)
    keep = (xb > t) | ((xb == t) & (col <= ir))
    o_ref[...] = jnp.where(keep, jnp.maximum(xb, 0.0), 0.0)


def kernel(x, k):
    del k
    packed = _sc_thresholds(x)
    q = packed[:, :8].reshape(_R, 2)
    tf = lax.bitcast_convert_type(q[:, 0], jnp.float32).reshape(_R, 1)
    ir = q[:, 1].reshape(_R, 1)

    BR = 32
    return pl.pallas_call(
        _mask_body,
        grid=(_R // BR,),
        in_specs=[
            pl.BlockSpec((BR, _N), lambda i: (i, 0)),
            pl.BlockSpec((BR, 1), lambda i: (i, 0)),
            pl.BlockSpec((BR, 1), lambda i: (i, 0)),
        ],
        out_specs=pl.BlockSpec((BR, _N), lambda i: (i, 0)),
        out_shape=jax.ShapeDtypeStruct(x.shape, x.dtype),
    )(x, tf, ir)


# speedup vs baseline: 5.5410x; 1.0500x over previous
import functools

import jax
import jax.numpy as jnp
import numpy as np
from jax import lax
from jax.experimental import pallas as pl
from jax.experimental.pallas import tpu as pltpu
from jax.experimental.pallas import tpu_sc as plsc

_K = 32
_BIG_IDX = np.int32(2147483647)

_R, _N = 128, 32768
_NC, _NS = 2, 16
_NW = _NC * _NS
_RPW = _R // _NW
_NV = _N // 16


def _sort16d(v):
    return -jnp.sort(-v)


def _merge32(a0, a1):
    rb = lax.rev(a1, (0,))
    hi = jnp.maximum(a0, rb)
    lo = jnp.minimum(a0, rb)
    return _sort16d(hi), _sort16d(lo)


def _top32_fold(acc, sv):
    a0, a1 = acc
    m0, _ = _merge32(a1, sv)
    return _merge32(a0, m0)


def _sc_body(x_hbm, out_hbm, buf0, buf1, idxb, gmax_s, res_v, sem0, sem1,
             rsem):
    wid = lax.axis_index("s") * _NC + lax.axis_index("c")
    base = wid * _RPW
    iota = lax.iota(jnp.int32, 16)
    ninf = jnp.full((16,), -jnp.inf, jnp.float32)

    bufs = (buf0, buf1)
    sems = (sem0, sem1)
    buf0[pl.ds(_N, 16)] = ninf
    buf1[pl.ds(_N, 16)] = ninf

    pending = pltpu.async_copy(x_hbm.at[base], buf0.at[pl.ds(0, _N)], sem0)
    res = jnp.zeros((16,), jnp.int32)

    for j in range(_RPW):
        row_v = bufs[j % 2]
        pending.wait()
        if j + 1 < _RPW:
            pending = pltpu.async_copy(
                x_hbm.at[base + j + 1],
                bufs[(j + 1) % 2].at[pl.ds(0, _N)], sems[(j + 1) % 2])

        def p1(g, accs, row_v=row_v):
            vs = [row_v[pl.ds(g * 128 + u * 16, 16)] for u in range(8)]
            f01 = jnp.maximum(vs[0], vs[1])
            f23 = jnp.maximum(vs[2], vs[3])
            f45 = jnp.maximum(vs[4], vs[5])
            f67 = jnp.maximum(vs[6], vs[7])
            gmax_s[g] = jnp.max(jnp.maximum(jnp.maximum(f01, f23),
                                            jnp.maximum(f45, f67)))
            return tuple(jnp.maximum(accs[u], vs[u]) for u in range(8))

        accs = lax.fori_loop(0, _N // 128, p1, (ninf,) * 8)
        sacc = _merge32(_sort16d(accs[0]), _sort16d(accs[1]))
        for u in range(2, 8):
            sacc = _top32_fold(sacc, _sort16d(accs[u]))
        s = sacc[1][15]
        svec = jnp.full((16,), s, jnp.float32)

        def collect8(g, cntv, row_v=row_v, svec=svec):
            def c1(u, cv):
                v = row_v[pl.ds(g * 128 + u * 16, 16)]
                msk = v >= svec
                plsc.store_compressed(
                    idxb.at[pl.ds(cv[0], 16)],
                    iota + (g * 128 + u * 16), mask=msk)
                return cv + plsc.all_reduce_population_count(msk)

            return lax.fori_loop(0, 8, c1, cntv)

        def p2(b, cntv, s=s, collect8=collect8):
            g0 = b * 4
            hit = ((gmax_s[g0] >= s) | (gmax_s[g0 + 1] >= s) |
                   (gmax_s[g0 + 2] >= s) | (gmax_s[g0 + 3] >= s))

            def slow(cv):
                for u in range(4):
                    cv = lax.cond(gmax_s[g0 + u] >= s,
                                  lambda c, gu=g0 + u: collect8(gu, c),
                                  lambda c: c, cv)
                return cv

            return lax.cond(hit, slow, lambda cv: cv, cntv)

        cntv = lax.fori_loop(0, _N // 512, p2, jnp.zeros((16,), jnp.int32))
        cnt = cntv[0]
        idxb[pl.ds(cnt, 16)] = jnp.full((16,), _N, jnp.int32)
        nvq = (cnt + 15) // 16

        def mstep(q, acc, row_v=row_v):
            iq = idxb[pl.ds(q * 16, 16)]
            vq = plsc.load_gather(row_v, [iq])
            return _top32_fold(acc, _sort16d(vq))

        acc0, acc1 = lax.fori_loop(0, nvq, mstep, (ninf, ninf))
        tval = acc1[15]
        tv = jnp.full((16,), tval, jnp.float32)

        def cgt(q, c, row_v=row_v):
            iq = idxb[pl.ds(q * 16, 16)]
            vq = plsc.load_gather(row_v, [iq])
            return c + plsc.all_reduce_population_count(vq > tv)

        r = _K - lax.fori_loop(0, nvq, cgt,
                               jnp.zeros((16,), jnp.int32))[0]

        def min_eq_idx(_, row_v=row_v, nvq=nvq):
            def fmin(q, acc):
                iq = idxb[pl.ds(q * 16, 16)]
                vq = plsc.load_gather(row_v, [iq])
                return jnp.minimum(acc, jnp.where(vq == tv, iq, _BIG_IDX))

            mi = lax.fori_loop(0, nvq, fmin,
                               jnp.full((16,), _BIG_IDX, jnp.int32))
            return -jnp.max(-mi)

        def rth_eq_idx(_, row_v=row_v, nvq=nvq, r=r):
            def bstep(b, ans):
                cand = ans | (jnp.int32(1) << (14 - b))
                cv = jnp.full((16,), cand, jnp.int32)

                def fcnt(q, c):
                    iq = idxb[pl.ds(q * 16, 16)]
                    vq = plsc.load_gather(row_v, [iq])
                    return c + plsc.all_reduce_population_count(
                        (vq == tv) & (iq < cv))

                cnt = lax.fori_loop(0, nvq, fcnt,
                                    jnp.zeros((16,), jnp.int32))[0]
                return jnp.where(cnt < r, cand, ans)

            return lax.fori_loop(0, 15, bstep, jnp.int32(0))

        tidx = lax.cond(r == 1, min_eq_idx, rth_eq_idx, 0)
        tbits = lax.bitcast_convert_type(tval, jnp.int32)
        res = jnp.where(iota == 2 * j, jnp.full((16,), tbits, jnp.int32),
                        res)
        res = jnp.where(iota == 2 * j + 1, jnp.full((16,), tidx, jnp.int32),
                        res)

    res_v[...] = res
    pltpu.async_copy(res_v, out_hbm.at[wid], rsem).wait()


_sc_thresholds = functools.partial(
    pl.kernel,
    out_type=jax.ShapeDtypeStruct((_NW, 16), jnp.int32),
    mesh=plsc.VectorSubcoreMesh(core_axis_name="c", subcore_axis_name="s"),
    compiler_params=pltpu.CompilerParams(needs_layout_passes=False),
    scratch_types=[
        pltpu.VMEM((_N + 16,), jnp.float32),
        pltpu.VMEM((_N + 16,), jnp.float32),
        pltpu.VMEM((_N + 16,), jnp.int32),
        pltpu.SMEM((_N // 128,), jnp.float32),
        pltpu.VMEM((16,), jnp.int32),
        pltpu.SemaphoreType.DMA,
        pltpu.SemaphoreType.DMA,
        pltpu.SemaphoreType.DMA,
    ],
)(_sc_body)


def _mask_body(x_ref, t_ref, i_ref, o_ref):
    xb = x_ref[...]
    t = t_ref[...]
    ir = i_ref[...]
    col = lax.broadcasted_iota(jnp.int32, xb.shape, 1)
    keep = (xb > t) | ((xb == t) & (col <= ir))
    o_ref[...] = jnp.where(keep, jnp.maximum(xb, 0.0), 0.0)


def kernel(x, k):
    del k
    packed = _sc_thresholds(x)
    q = packed[:, :8].reshape(_R, 2)
    tf = lax.bitcast_convert_type(q[:, 0], jnp.float32).reshape(_R, 1)
    ir = q[:, 1].reshape(_R, 1)

    BR = 32
    return pl.pallas_call(
        _mask_body,
        grid=(_R // BR,),
        in_specs=[
            pl.BlockSpec((BR, _N), lambda i: (i, 0)),
            pl.BlockSpec((BR, 1), lambda i: (i, 0)),
            pl.BlockSpec((BR, 1), lambda i: (i, 0)),
        ],
        out_specs=pl.BlockSpec((BR, _N), lambda i: (i, 0)),
        out_shape=jax.ShapeDtypeStruct(x.shape, x.dtype),
    )(x, tf, ir)
